# Initial kernel scaffold; baseline (speedup 1.0000x reference)
import jax
import jax.numpy as jnp
from jax.experimental import pallas as pl


def kernel(src, weight):
    raise NotImplementedError("write your pallas kernel here")


# speedup vs baseline: 1.4571x; 1.4571x over previous
import functools

import jax
import jax.numpy as jnp
from jax import lax
from jax.experimental import pallas as pl
from jax.experimental.pallas import tpu as pltpu
from jax.experimental.pallas import tpu_sc as plsc


def _build_gather(N, D, NC, NS, CH):
    NW = NC * NS
    b_per_w = N // NW
    n_ch = b_per_w // CH
    mesh = plsc.VectorSubcoreMesh(core_axis_name="c", subcore_axis_name="s")

    @functools.partial(
        pl.kernel,
        mesh=mesh,
        out_type=jax.ShapeDtypeStruct((N, D), jnp.float32),
        scratch_types=[
            pltpu.VMEM((CH,), jnp.int32),
            pltpu.VMEM((CH, D), jnp.float32),
            pltpu.SemaphoreType.DMA,
        ],
        compiler_params=pltpu.CompilerParams(use_tc_tiling_on_sc=False),
    )
    def gather(idx_hbm, tbl_hbm, out_hbm, idx_v, rows_v, sem):
        wid = lax.axis_index("s") * NC + lax.axis_index("c")
        base = wid * b_per_w

        def body(i, carry):
            off = base + i * CH
            pltpu.sync_copy(idx_hbm.at[pl.ds(off, CH)], idx_v)
            pltpu.async_copy(tbl_hbm.at[idx_v], rows_v, sem).wait()
            pltpu.sync_copy(rows_v, out_hbm.at[pl.ds(off, CH)])
            return carry

        lax.fori_loop(0, n_ch, body, 0)

    return gather


def kernel(src, weight):
    B, H = src.shape
    V, D = weight.shape
    N = B * H
    idx = src.reshape(N)
    info = plsc.get_sparse_core_info()
    gather = _build_gather(N, D, info.num_cores, info.num_subcores, 1024)
    out = gather(idx, weight)
    return out.reshape(B, H, D)


# speedup vs baseline: 1.5006x; 1.0299x over previous
import functools

import jax
import jax.numpy as jnp
from jax import lax
from jax.experimental import pallas as pl
from jax.experimental.pallas import tpu as pltpu
from jax.experimental.pallas import tpu_sc as plsc


def _build_gather(N, D, NC, NS, CH):
    NW = NC * NS
    b_per_w = N // NW
    n_ch = b_per_w // CH
    assert n_ch >= 4 and n_ch % 2 == 0
    mesh = plsc.VectorSubcoreMesh(core_axis_name="c", subcore_axis_name="s")

    @functools.partial(
        pl.kernel,
        mesh=mesh,
        out_type=jax.ShapeDtypeStruct((N, D), jnp.float32),
        scratch_types=[
            pltpu.VMEM((2, CH), jnp.int32),
            pltpu.VMEM((2, CH, D), jnp.float32),
            pltpu.SemaphoreType.DMA,
            pltpu.SemaphoreType.DMA,
            pltpu.SemaphoreType.DMA,
            pltpu.SemaphoreType.DMA,
            pltpu.SemaphoreType.DMA,
            pltpu.SemaphoreType.DMA,
        ],
        compiler_params=pltpu.CompilerParams(use_tc_tiling_on_sc=False),
    )
    def gather(idx_hbm, tbl_hbm, out_hbm, idx_v, rows_v,
               si0, si1, sg0, sg1, sw0, sw1):
        sem_i = (si0, si1)
        sem_g = (sg0, sg1)
        sem_w = (sw0, sw1)
        wid = lax.axis_index("s") * NC + lax.axis_index("c")
        base = wid * b_per_w

        def issue_idx(i, b):
            pltpu.async_copy(
                idx_hbm.at[pl.ds(base + i * CH, CH)], idx_v.at[b], sem_i[b])

        def wait_idx(b):
            pltpu.make_async_copy(
                idx_hbm.at[pl.ds(base, CH)], idx_v.at[b], sem_i[b]).wait()

        def issue_gather(b):
            pltpu.async_copy(tbl_hbm.at[idx_v.at[b]], rows_v.at[b], sem_g[b])

        def wait_gather(b):
            pltpu.make_async_copy(
                tbl_hbm.at[idx_v.at[b]], rows_v.at[b], sem_g[b]).wait()

        def issue_write(i, b):
            pltpu.async_copy(
                rows_v.at[b], out_hbm.at[pl.ds(base + i * CH, CH)], sem_w[b])

        def wait_write(b):
            pltpu.make_async_copy(
                rows_v.at[b], out_hbm.at[pl.ds(base, CH)], sem_w[b]).wait()

        for b in range(2):
            issue_idx(b, b)
        for b in range(2):
            wait_idx(b)
            issue_gather(b)

        def body(k, carry):
            for b in range(2):
                i = 2 * k + b
                wait_gather(b)
                issue_write(i, b)
                issue_idx(i + 2, b)
                wait_idx(b)
                wait_write(b)
                issue_gather(b)
            return carry

        lax.fori_loop(0, (n_ch - 2) // 2, body, 0)

        for b in range(2):
            i = n_ch - 2 + b
            wait_gather(b)
            issue_write(i, b)
        for b in range(2):
            wait_write(b)

    return gather


def kernel(src, weight):
    B, H = src.shape
    V, D = weight.shape
    N = B * H
    idx = src.reshape(N)
    info = plsc.get_sparse_core_info()
    gather = _build_gather(N, D, info.num_cores, info.num_subcores, 1600)
    out = gather(idx, weight)
    return out.reshape(B, H, D)


# speedup vs baseline: 1.5013x; 1.0005x over previous
import functools

import jax
import jax.numpy as jnp
from jax import lax
from jax.experimental import pallas as pl
from jax.experimental.pallas import tpu as pltpu
from jax.experimental.pallas import tpu_sc as plsc

_NB = 4
_CH = 800


def _build_gather(N, D, NC, NS, CH, NB):
    NW = NC * NS
    b_per_w = N // NW
    n_ch = b_per_w // CH
    assert n_ch >= 2 * NB and n_ch % NB == 0
    mesh = plsc.VectorSubcoreMesh(core_axis_name="c", subcore_axis_name="s")

    @functools.partial(
        pl.kernel,
        mesh=mesh,
        out_type=jax.ShapeDtypeStruct((N, D), jnp.float32),
        scratch_types=[
            pltpu.VMEM((NB, CH), jnp.int32),
            pltpu.VMEM((NB, CH, D), jnp.float32),
        ] + [pltpu.SemaphoreType.DMA] * (3 * NB),
        compiler_params=pltpu.CompilerParams(use_tc_tiling_on_sc=False),
    )
    def gather(idx_hbm, tbl_hbm, out_hbm, idx_v, rows_v, *sems):
        sem_i = sems[0:NB]
        sem_g = sems[NB:2 * NB]
        sem_w = sems[2 * NB:3 * NB]
        wid = lax.axis_index("s") * NC + lax.axis_index("c")
        base = wid * b_per_w

        def issue_idx(i, b):
            pltpu.async_copy(
                idx_hbm.at[pl.ds(base + i * CH, CH)], idx_v.at[b], sem_i[b])

        def wait_idx(b):
            pltpu.make_async_copy(
                idx_hbm.at[pl.ds(base, CH)], idx_v.at[b], sem_i[b]).wait()

        def issue_gather(b):
            pltpu.async_copy(tbl_hbm.at[idx_v.at[b]], rows_v.at[b], sem_g[b])

        def wait_gather(b):
            pltpu.make_async_copy(
                tbl_hbm.at[idx_v.at[b]], rows_v.at[b], sem_g[b]).wait()

        def issue_write(i, b):
            pltpu.async_copy(
                rows_v.at[b], out_hbm.at[pl.ds(base + i * CH, CH)], sem_w[b])

        def wait_write(b):
            pltpu.make_async_copy(
                rows_v.at[b], out_hbm.at[pl.ds(base, CH)], sem_w[b]).wait()

        for b in range(NB):
            issue_idx(b, b)
        for b in range(NB):
            wait_idx(b)
            issue_gather(b)

        def body(k, carry):
            for b in range(NB):
                i = NB * k + b
                wait_gather(b)
                issue_write(i, b)
                issue_idx(i + NB, b)
                wait_idx(b)
                wait_write(b)
                issue_gather(b)
            return carry

        lax.fori_loop(0, (n_ch - NB) // NB, body, 0)

        for b in range(NB):
            i = n_ch - NB + b
            wait_gather(b)
            issue_write(i, b)
        for b in range(NB):
            wait_write(b)

    return gather


def kernel(src, weight):
    B, H = src.shape
    V, D = weight.shape
    N = B * H
    idx = src.reshape(N)
    info = plsc.get_sparse_core_info()
    gather = _build_gather(N, D, info.num_cores, info.num_subcores, _CH, _NB)
    out = gather(idx, weight)
    return out.reshape(B, H, D)


# speedup vs baseline: 1.5015x; 1.0001x over previous
import functools

import jax
import jax.numpy as jnp
from jax import lax
from jax.experimental import pallas as pl
from jax.experimental.pallas import tpu as pltpu
from jax.experimental.pallas import tpu_sc as plsc

_NB = 4
_CH = 640


def _build_gather(N, D, NC, NS, CH, NB):
    NW = NC * NS
    b_per_w = N // NW
    n_ch = b_per_w // CH
    assert n_ch >= 2 * NB and n_ch % NB == 0
    mesh = plsc.VectorSubcoreMesh(core_axis_name="c", subcore_axis_name="s")

    @functools.partial(
        pl.kernel,
        mesh=mesh,
        out_type=jax.ShapeDtypeStruct((N, D), jnp.float32),
        scratch_types=[
            pltpu.VMEM((b_per_w,), jnp.int32),
            pltpu.VMEM((NB, CH, D), jnp.float32),
            pltpu.SemaphoreType.DMA,
        ] + [pltpu.SemaphoreType.DMA] * (2 * NB),
        compiler_params=pltpu.CompilerParams(use_tc_tiling_on_sc=False),
    )
    def gather(idx_hbm, tbl_hbm, out_hbm, idx_v, rows_v, sem_idx, *sems):
        sem_g = sems[0:NB]
        sem_w = sems[NB:2 * NB]
        wid = lax.axis_index("s") * NC + lax.axis_index("c")
        base = wid * b_per_w

        def issue_gather(i, b):
            pltpu.async_copy(
                tbl_hbm.at[idx_v.at[pl.ds(i * CH, CH)]], rows_v.at[b],
                sem_g[b])

        def wait_gather(b):
            pltpu.make_async_copy(
                tbl_hbm.at[idx_v.at[pl.ds(0, CH)]], rows_v.at[b],
                sem_g[b]).wait()

        def issue_write(i, b):
            pltpu.async_copy(
                rows_v.at[b], out_hbm.at[pl.ds(base + i * CH, CH)], sem_w[b])

        def wait_write(b):
            pltpu.make_async_copy(
                rows_v.at[b], out_hbm.at[pl.ds(base, CH)], sem_w[b]).wait()

        pltpu.async_copy(
            idx_hbm.at[pl.ds(base, b_per_w)], idx_v, sem_idx).wait()

        for b in range(NB):
            issue_gather(b, b)

        def body(k, carry):
            for b in range(NB):
                i = NB * k + b
                wait_gather(b)
                issue_write(i, b)
                wait_write(b)
                issue_gather(i + NB, b)
            return carry

        lax.fori_loop(0, (n_ch - NB) // NB, body, 0)

        for b in range(NB):
            i = n_ch - NB + b
            wait_gather(b)
            issue_write(i, b)
        for b in range(NB):
            wait_write(b)

    return gather


def kernel(src, weight):
    B, H = src.shape
    V, D = weight.shape
    N = B * H
    idx = src.reshape(N)
    info = plsc.get_sparse_core_info()
    gather = _build_gather(N, D, info.num_cores, info.num_subcores, _CH, _NB)
    out = gather(idx, weight)
    return out.reshape(B, H, D)
